# Initial kernel scaffold; baseline (speedup 1.0000x reference)
import jax
import jax.numpy as jnp
from jax.experimental import pallas as pl


def kernel(x):
    raise NotImplementedError("write your pallas kernel here")


# speedup vs baseline: 7.4869x; 7.4869x over previous
import jax
import jax.numpy as jnp
from jax.experimental import pallas as pl
from jax.experimental.pallas import tpu as pltpu

_K = 8
_TP = 112


def _oam_body(xc_ref, out_ref, d_ref):
    nch = xc_ref.shape[1]
    p = xc_ref.shape[3]
    nt = p // _TP

    s = jnp.zeros((1, p), jnp.float32)
    for t in range(nt):
        rp0 = t * _TP

        def body(ci, acc):
            chunk = xc_ref[0, ci]
            part = chunk[:, rp0:rp0 + _TP]
            cols = part.T
            for k in range(_K):
                acc = acc + jnp.abs(cols[:, k:k + 1] - chunk[k:k + 1, :])
            return acc

        acc = jax.lax.fori_loop(0, nch, body,
                                jnp.zeros((_TP, p), jnp.float32))
        d_ref[rp0:rp0 + _TP, :] = acc
        s = s + jnp.sum(acc, axis=0, keepdims=True)

    r = 1.0 / jnp.maximum(s, 1e-12)
    u = jnp.full((8, p), 1.0 / p, jnp.float32)
    for _ in range(10):
        acc_u = jnp.zeros((8, p), jnp.float32)
        for t in range(nt):
            rp0 = t * _TP
            acc_u = acc_u + jnp.dot(u[:, rp0:rp0 + _TP],
                                    d_ref[rp0:rp0 + _TP, :],
                                    preferred_element_type=jnp.float32)
        u = acc_u * r
    out_ref[0] = u


def kernel(x):
    b, c, h, w = x.shape
    p = h * w
    xc = x.reshape(b, c // _K, _K, p)
    out = pl.pallas_call(
        _oam_body,
        grid=(b,),
        in_specs=[pl.BlockSpec((1, c // _K, _K, p), lambda i: (i, 0, 0, 0))],
        out_specs=pl.BlockSpec((1, 8, p), lambda i: (i, 0, 0)),
        out_shape=jax.ShapeDtypeStruct((b, 8, p), jnp.float32),
        scratch_shapes=[pltpu.VMEM((p, p), jnp.float32)],
        compiler_params=pltpu.CompilerParams(
            dimension_semantics=("parallel",),
            vmem_limit_bytes=64 * 1024 * 1024,
        ),
    )(xc)
    return out[:, 0, :].reshape(b, h, w)
